# Initial kernel scaffold; baseline (speedup 1.0000x reference)
import jax
import jax.numpy as jnp
from jax.experimental import pallas as pl


def kernel(x, edge_index, edge_weight, W0, b0, Ww0, bw0, W1, b1, Ww1, bw1, gamma, beta, damping):
    raise NotImplementedError("write your pallas kernel here")


# speedup vs baseline: 3.4114x; 3.4114x over previous
import functools

import jax
import jax.numpy as jnp
from jax import lax
from jax.experimental import pallas as pl
from jax.experimental.pallas import tpu as pltpu
from jax.experimental.pallas import tpu_sc as plsc

N = 4096
E = 131072
D = 256
ED = 16

NW = 32
EPW = E // NW
IDX_ROWS = EPW // 128
NSUB = 16
ROWS_PER_SUB = N // NSUB


AWR = 17
CHUNK = 512
NCHUNK = EPW // CHUNK


def _sc_scatter_body(src_hbm, ewt_hbm, z_hbm, acc_out,
                     idx_v, ewt_v, acc_v):
    cid = lax.axis_index("c")
    sid = lax.axis_index("s")
    wid = cid * NSUB + sid

    pltpu.sync_copy(z_hbm, acc_v)
    pltpu.sync_copy(src_hbm.at[pl.ds(wid * EPW, EPW)], idx_v)

    def _chunk(k, _):
        pltpu.sync_copy(
            ewt_hbm.at[pl.ds((wid * NCHUNK + k) * AWR * CHUNK, AWR * CHUNK)],
            ewt_v)

        def _grp(g, _):
            idx16 = idx_v[pl.ds(k * CHUNK + g * 16, 16)]
            for c in range(AWR):
                plsc.addupdate_scatter(acc_v.at[pl.ds(c * N, N)], [idx16],
                                       ewt_v[pl.ds(c * CHUNK + g * 16, 16)])
            return 0

        lax.fori_loop(0, CHUNK // 16, _grp, 0)
        return 0

    lax.fori_loop(0, NCHUNK, _chunk, 0)
    pltpu.sync_copy(acc_v, acc_out.at[pl.ds(wid * AWR * N, AWR * N)])


def _sc_scatter(src1d, ewt_flat, zeros_cm):
    mesh = plsc.VectorSubcoreMesh(core_axis_name="c", subcore_axis_name="s")
    k = pl.kernel(
        _sc_scatter_body,
        mesh=mesh,
        out_type=jax.ShapeDtypeStruct((NW * AWR * N,), jnp.float32),
        scratch_types=[
            pltpu.VMEM((EPW,), jnp.int32),
            pltpu.VMEM((AWR * CHUNK,), jnp.float32),
            pltpu.VMEM((AWR * N,), jnp.float32),
        ],
        compiler_params=pltpu.CompilerParams(needs_layout_passes=False),
    )
    return k(src1d, ewt_flat, zeros_cm)


def _sc_rsrc_body(r_hbm, src_hbm, out_hbm, r_v, idx_v, rs_v):
    cid = lax.axis_index("c")
    sid = lax.axis_index("s")
    wid = cid * NSUB + sid
    pltpu.sync_copy(r_hbm, r_v)
    pltpu.sync_copy(src_hbm.at[pl.ds(wid * EPW, EPW)], idx_v)

    def _g(i, _):
        idx = idx_v[pl.ds(i * 16, 16)]
        rs_v[pl.ds(i * 16, 16)] = plsc.load_gather(r_v, [idx])
        return 0

    lax.fori_loop(0, EPW // 16, _g, 0)
    pltpu.sync_copy(rs_v, out_hbm.at[pl.ds(wid * EPW, EPW)])


def _sc_rsrc(r1d, src1d):
    mesh = plsc.VectorSubcoreMesh(core_axis_name="c", subcore_axis_name="s")
    k = pl.kernel(
        _sc_rsrc_body,
        mesh=mesh,
        out_type=jax.ShapeDtypeStruct((E,), jnp.float32),
        scratch_types=[
            pltpu.VMEM((N,), jnp.float32),
            pltpu.VMEM((EPW,), jnp.int32),
            pltpu.VMEM((EPW,), jnp.float32),
        ],
        compiler_params=pltpu.CompilerParams(needs_layout_passes=False),
    )
    return k(r1d, src1d)


NI = 16
BI = N // NI


def _pre_body(x_ref, w0_ref, wseg_ref, b0_ref, bw0d_ref, w1_ref,
              acc_ref,
              u_ref, v_ref, r_ref, q_ref, sew_ref, srs_ref):
    i = pl.program_id(0)
    accs = jnp.sum(acc_ref[...], axis=0)
    ra = lax.broadcasted_iota(jnp.int32, (BI, BI), 0)
    ca = lax.broadcasted_iota(jnp.int32, (BI, BI), 1)
    ident = jnp.where(ra == ca, 1.0, 0.0)
    seg = lax.dot_general(ident, accs[0:ED, :], (((1,), (1,)), ((), ())),
                          preferred_element_type=jnp.float32)
    cnt = lax.dot_general(ident, accs[ED:ED + 1, :], (((1,), (1,)), ((), ())),
                          preferred_element_type=jnp.float32)
    out0 = lax.dot_general(x_ref[...], w0_ref[...],
                           (((1,), (1,)), ((), ())),
                           preferred_element_type=jnp.float32)
    out0 = out0 + lax.dot_general(seg, wseg_ref[...],
                                  (((1,), (1,)), ((), ())),
                                  preferred_element_type=jnp.float32)
    out0 = out0 + b0_ref[...] + cnt * bw0d_ref[...]
    r = jnp.sum(out0, axis=1, keepdims=True)
    r_ref[...] = r
    nrm = jax.lax.rsqrt(jnp.sum(out0 * out0, axis=1, keepdims=True))
    u_ref[...] = out0 * nrm
    v_ref[...] = lax.dot_general(out0, w1_ref[...],
                                 (((1,), (1,)), ((), ())),
                                 preferred_element_type=jnp.float32)

    @pl.when(i == 0)
    def _():
        q_ref[...] = jnp.zeros_like(q_ref)
        sew_ref[...] = jnp.zeros_like(sew_ref)
        srs_ref[...] = jnp.zeros_like(srs_ref)

    q_ref[...] += lax.dot_general(r, seg, (((0,), (0,)), ((), ())),
                                  preferred_element_type=jnp.float32)
    ones_row = jnp.ones((1, BI), jnp.float32)
    sew_ref[...] += lax.dot_general(ones_row, seg, (((1,), (0,)), ((), ())),
                                    preferred_element_type=jnp.float32)
    sr = lax.dot_general(cnt, r, (((0,), (0,)), ((), ())),
                         preferred_element_type=jnp.float32)
    srr = lax.dot_general(cnt, r * r, (((0,), (0,)), ((), ())),
                          preferred_element_type=jnp.float32)
    lane = lax.broadcasted_iota(jnp.int32, (1, ED), 1)
    srs_ref[...] += (jnp.where(lane == 0, sr, 0.0)
                     + jnp.where(lane == 1, srr, 0.0))


def _pre(x, W0, Wseg, b0r, bw0d, W1, acc_part, interpret=False):
    two16 = pl.BlockSpec((1, ED), lambda i: (0, 0))
    return pl.pallas_call(
        _pre_body,
        grid=(NI,),
        in_specs=[
            pl.BlockSpec((BI, D), lambda i: (i, 0)),
            pl.BlockSpec((D, D), lambda i: (0, 0)),
            pl.BlockSpec((D, ED), lambda i: (0, 0)),
            pl.BlockSpec((1, D), lambda i: (0, 0)),
            pl.BlockSpec((1, D), lambda i: (0, 0)),
            pl.BlockSpec((D, D), lambda i: (0, 0)),
            pl.BlockSpec((NW, AWR, BI), lambda i: (0, 0, i)),
        ],
        out_specs=[
            pl.BlockSpec((BI, D), lambda i: (i, 0)),
            pl.BlockSpec((BI, D), lambda i: (i, 0)),
            pl.BlockSpec((BI, 1), lambda i: (i, 0)),
            two16, two16, two16,
        ],
        out_shape=[
            jax.ShapeDtypeStruct((N, D), jnp.float32),
            jax.ShapeDtypeStruct((N, D), jnp.float32),
            jax.ShapeDtypeStruct((N, 1), jnp.float32),
            jax.ShapeDtypeStruct((1, ED), jnp.float32),
            jax.ShapeDtypeStruct((1, ED), jnp.float32),
            jax.ShapeDtypeStruct((1, ED), jnp.float32),
        ],
        compiler_params=pltpu.CompilerParams(
            dimension_semantics=("arbitrary",)),
        interpret=interpret,
    )(x, W0, Wseg, b0r, bw0d, W1, acc_part)


EBLK = 4096
NEB = E // EBLK


def _gram_body(ew_ref, g_ref):
    @pl.when(pl.program_id(0) == 0)
    def _():
        g_ref[...] = jnp.zeros_like(g_ref)

    blk = ew_ref[...]
    g_ref[...] += lax.dot_general(blk, blk, (((0,), (0,)), ((), ())),
                                  preferred_element_type=jnp.float32)


def _gram(edge_weight, interpret=False):
    return pl.pallas_call(
        _gram_body,
        grid=(NEB,),
        in_specs=[pl.BlockSpec((EBLK, ED), lambda i: (i, 0))],
        out_specs=pl.BlockSpec((ED, ED), lambda i: (0, 0)),
        out_shape=jax.ShapeDtypeStruct((ED, ED), jnp.float32),
        compiler_params=pltpu.CompilerParams(
            dimension_semantics=("arbitrary",)),
        interpret=interpret,
    )(edge_weight)


def _wout_body(ew_ref, m_ref, rs_ref, c_ref, v_ref, o_ref):
    o_ref[...] = (lax.dot_general(ew_ref[...], m_ref[...],
                                  (((1,), (0,)), ((), ())),
                                  preferred_element_type=jnp.float32)
                  + c_ref[...] + rs_ref[...] * v_ref[...])


def _wout(edge_weight, M, rsrc, c2, v2, interpret=False):
    return pl.pallas_call(
        _wout_body,
        grid=(NEB,),
        in_specs=[
            pl.BlockSpec((EBLK, ED), lambda i: (i, 0)),
            pl.BlockSpec((ED, ED), lambda i: (0, 0)),
            pl.BlockSpec((EBLK, 1), lambda i: (i, 0)),
            pl.BlockSpec((1, ED), lambda i: (0, 0)),
            pl.BlockSpec((1, ED), lambda i: (0, 0)),
        ],
        out_specs=pl.BlockSpec((EBLK, ED), lambda i: (i, 0)),
        out_shape=jax.ShapeDtypeStruct((E, ED), jnp.float32),
        compiler_params=pltpu.CompilerParams(
            dimension_semantics=("parallel",)),
        interpret=interpret,
    )(edge_weight, M, rsrc, c2, v2)


NJ = 16
BJ = N // NJ


def _attn_body(ui_ref, uj_ref, vj_ref, b1_ref, o_ref, acc_ref, den_ref):
    i = pl.program_id(0)
    j = pl.program_id(1)

    @pl.when(j == 0)
    def _():
        acc_ref[...] = jnp.zeros_like(acc_ref)
        den_ref[...] = jnp.zeros_like(den_ref)

    s = lax.dot_general(ui_ref[...], uj_ref[...], (((1,), (1,)), ((), ())),
                        preferred_element_type=jnp.float32)
    ra = lax.broadcasted_iota(jnp.int32, (BI, BJ), 0)
    ca = lax.broadcasted_iota(jnp.int32, (BI, BJ), 1)
    diag = jnp.where((ra == ca) & (i == j), 1.0, 0.0)
    e = jnp.exp(s - diag)
    den_ref[:, 0:1] += jnp.sum(e, axis=1, keepdims=True)
    acc_ref[...] += lax.dot_general(e, vj_ref[...], (((1,), (0,)), ((), ())),
                                    preferred_element_type=jnp.float32)

    @pl.when(j == NJ - 1)
    def _():
        o_ref[...] = acc_ref[...] / den_ref[:, 0:1] + b1_ref[...]


def _attn(u, V, b1r, interpret=False):
    return pl.pallas_call(
        _attn_body,
        grid=(NI, NJ),
        in_specs=[
            pl.BlockSpec((BI, D), lambda i, j: (i, 0)),
            pl.BlockSpec((BJ, D), lambda i, j: (j, 0)),
            pl.BlockSpec((BJ, D), lambda i, j: (j, 0)),
            pl.BlockSpec((1, D), lambda i, j: (0, 0)),
        ],
        out_specs=pl.BlockSpec((BI, D), lambda i, j: (i, 0)),
        out_shape=jax.ShapeDtypeStruct((N, D), jnp.float32),
        scratch_shapes=[
            pltpu.VMEM((BI, D), jnp.float32),
            pltpu.VMEM((BI, 128), jnp.float32),
        ],
        compiler_params=pltpu.CompilerParams(
            dimension_semantics=("arbitrary", "arbitrary")),
        interpret=interpret,
    )(u, u, V, b1r)


def kernel(x, edge_index, edge_weight, W0, b0, Ww0, bw0, W1, b1, Ww1, bw1,
           gamma, beta, damping):
    d = damping.astype(jnp.float32)
    src1d = edge_index[0]
    ew_aug = jnp.concatenate(
        [edge_weight, jnp.ones((E, 1), jnp.float32)], axis=1)
    ewt_flat = (ew_aug.reshape(NW, NCHUNK, CHUNK, AWR)
                .transpose(0, 1, 3, 2)
                .reshape(NW * NCHUNK * AWR * CHUNK))

    acc_flat = _sc_scatter(src1d, ewt_flat,
                           jnp.zeros((AWR * N,), jnp.float32))
    acc_part = acc_flat.reshape(NW, AWR, N)

    u, V, r, q, sew, srs = _pre(x, W0, d * Ww0, b0.reshape(1, D),
                                (d * bw0).reshape(1, D), W1, acc_part)
    G = _gram(edge_weight)

    Sew = sew[0]
    qv = q[0]
    Sr = srs[0, 0]
    Srr = srs[0, 1]
    Sa = Sew @ Ww0.T + E * bw0
    mean = (Sa + d * Sr) / E
    Saa = jnp.einsum('jk,kl,jl->j', Ww0, G, Ww0) + 2 * bw0 * (Ww0 @ Sew) \
        + E * bw0 ** 2
    Sar = Ww0 @ qv + bw0 * Sr
    Sww = Saa + 2 * d * Sar + d * d * Srr
    var = Sww / E - mean ** 2
    s = gamma * jax.lax.rsqrt(var + 1e-5)
    t = beta - mean * s
    M = Ww0.T @ (s[:, None] * Ww1.T)
    c = (bw0 * s + t) @ Ww1.T + bw1
    v = d * (s @ Ww1.T)

    rsrc = _sc_rsrc(r.reshape(N), edge_index[0])
    out_w = _wout(edge_weight, M, rsrc.reshape(E, 1),
                  c.reshape(1, ED), v.reshape(1, ED))
    out1 = _attn(u, V, b1.reshape(1, D))
    return (out1, out_w)
